# baseline (speedup 1.0000x reference)
import functools

import jax
import jax.numpy as jnp
from jax import lax
from jax.experimental import pallas as pl
from jax.experimental.pallas import tpu as pltpu
from jax.experimental.pallas import tpu_sc as plsc

USER_DIM = 1000000
EMBED_DIM = 64
OUT_DIM = 64
BATCH = 16384

NC, NS = 2, 16
NW = NC * NS
B_PER_W = BATCH // NW
CHUNK = 128
N_CHUNKS = B_PER_W // CHUNK

N_TILES = (USER_DIM + 127) // 128
TILES_PER_W = (N_TILES + NW - 1) // NW
T2_ROWS = N_TILES * 64

_mesh = plsc.VectorSubcoreMesh(
    core_axis_name="c", subcore_axis_name="s", num_cores=NC, num_subcores=NS
)


@functools.partial(
    pl.kernel,
    out_type=jax.ShapeDtypeStruct((T2_ROWS, 128), jnp.float32),
    mesh=_mesh,
    scratch_types=[
        pltpu.VMEM((EMBED_DIM, 128), jnp.float32),
        pltpu.VMEM((EMBED_DIM, 128), jnp.float32),
    ],
    compiler_params=pltpu.CompilerParams(needs_layout_passes=False),
)
def _sc_transpose(tableT_hbm, tail_hbm, t2_hbm, chunk_v, out_v):
    wid = lax.axis_index("s") * NC + lax.axis_index("c")
    lane = lax.iota(jnp.int32, 16)

    def tile(k, carry):
        j = wid + k * NW

        @pl.when(j < N_TILES - 1)
        def _():
            col = pl.multiple_of(j * 128, 128)
            pltpu.sync_copy(tableT_hbm.at[:, pl.ds(col, 128)], chunk_v)

        @pl.when(j == N_TILES - 1)
        def _():
            pltpu.sync_copy(tail_hbm, chunk_v)

        @pl.when(j < N_TILES)
        def _():
            for m in range(64):
                lo = jnp.full((16,), m, jnp.int32)
                hi = jnp.full((16,), 64 + m, jnp.int32)
                for q in range(4):
                    rows = lane + 16 * q
                    out_v[m, pl.ds(16 * q, 16)] = plsc.load_gather(
                        chunk_v, [rows, lo]
                    )
                    out_v[m, pl.ds(64 + 16 * q, 16)] = plsc.load_gather(
                        chunk_v, [rows, hi]
                    )
            base = pl.multiple_of(j * 64, 64)
            pltpu.sync_copy(out_v, t2_hbm.at[pl.ds(base, 64)])

        return carry

    lax.fori_loop(0, TILES_PER_W, tile, 0)


@functools.partial(
    pl.kernel,
    out_type=jax.ShapeDtypeStruct((BATCH, 128), jnp.float32),
    mesh=_mesh,
    scratch_types=[
        pltpu.VMEM((N_CHUNKS, CHUNK), jnp.int32),
        pltpu.VMEM((B_PER_W, 128), jnp.float32),
        pltpu.SemaphoreType.DMA,
    ],
)
def _sc_gather(idx_hbm, table2_hbm, emb_hbm, idx_v, rows_v, sem):
    wid = lax.axis_index("s") * NC + lax.axis_index("c")
    base = wid * B_PER_W
    pltpu.sync_copy(idx_hbm.at[wid], idx_v)
    copies = [
        pltpu.async_copy(
            table2_hbm.at[idx_v.at[j]],
            rows_v.at[pl.ds(j * CHUNK, CHUNK)],
            sem,
        )
        for j in range(N_CHUNKS)
    ]
    for c in copies:
        c.wait()
    pltpu.sync_copy(rows_v, emb_hbm.at[pl.ds(base, B_PER_W)])


BM = 2048


def _mm_body(emb_ref, par_ref, ww_ref, b_ref, out_ref):
    lanes = lax.broadcasted_iota(jnp.int32, (BM, 128), 1)
    par = par_ref[...]
    sel = jnp.where(lanes < EMBED_DIM, 1.0 - par, par)
    out_ref[...] = (
        jnp.dot(emb_ref[...] * sel, ww_ref[...], preferred_element_type=jnp.float32)
        + b_ref[...]
    )


def _tc_linear(emb, par, WW, b2d):
    return pl.pallas_call(
        _mm_body,
        grid=(BATCH // BM,),
        in_specs=[
            pl.BlockSpec((BM, 128), lambda i: (i, 0)),
            pl.BlockSpec((BM, 1), lambda i: (i, 0)),
            pl.BlockSpec((128, OUT_DIM), lambda i: (0, 0)),
            pl.BlockSpec((1, OUT_DIM), lambda i: (0, 0)),
        ],
        out_specs=pl.BlockSpec((BM, OUT_DIM), lambda i: (i, 0)),
        out_shape=jax.ShapeDtypeStruct((BATCH, OUT_DIM), jnp.float32),
    )(emb, par, WW, b2d)


def kernel(x, table, W, b):
    xi = x.astype(jnp.int32)
    tableT = table.T
    tail = jnp.pad(
        lax.slice(tableT, (0, (N_TILES - 1) * 128), (EMBED_DIM, USER_DIM)),
        ((0, 0), (0, N_TILES * 128 - USER_DIM)),
    )
    table2 = _sc_transpose(tableT, tail)
    g = ((xi >> 7) << 6) | (xi & 63)
    idx2 = g.reshape(NW, N_CHUNKS, CHUNK)
    emb = _sc_gather(idx2, table2)
    par = ((xi >> 6) & 1).astype(jnp.float32).reshape(BATCH, 1)
    WW = jnp.concatenate([W, W], axis=0)
    return _tc_linear(emb, par, WW, b.reshape(1, OUT_DIM))


# speedup vs baseline: 3.3236x; 3.3236x over previous
import functools

import jax
import jax.numpy as jnp
from jax import lax
from jax.experimental import pallas as pl
from jax.experimental.pallas import tpu as pltpu
from jax.experimental.pallas import tpu_sc as plsc

USER_DIM = 1000000
EMBED_DIM = 64
OUT_DIM = 64
BATCH = 16384

NC, NS = 2, 16
NW = NC * NS
B_PER_W = BATCH // NW
CHUNK = 128
N_CHUNKS = B_PER_W // CHUNK

N_TILES = (USER_DIM + 127) // 128
T2_ROWS = N_TILES * 64
TPG = 4
N_GROUPS = (N_TILES - 1) // TPG
G_PER_W = (N_GROUPS + NW - 1) // NW

_mesh = plsc.VectorSubcoreMesh(
    core_axis_name="c", subcore_axis_name="s", num_cores=NC, num_subcores=NS
)


def _transpose_group(src, out_v, lane, n_tiles):
    def diag(d, carry):
        perm = jnp.where(lane + d < 16, lane + d, lane + d - 16)
        for s in range(n_tiles):
            for qp in range(4):
                for h in range(2):
                    for t in range(4):
                        v = plsc.load_gather(
                            src,
                            [16 * qp + lane, 128 * s + 64 * h + 16 * t + perm],
                        )
                        plsc.store_scatter(
                            out_v,
                            [64 * s + 16 * t + perm, 16 * qp + 64 * h + lane],
                            v,
                        )
        return carry

    lax.fori_loop(0, 16, diag, 0)


@functools.partial(
    pl.kernel,
    out_type=jax.ShapeDtypeStruct((T2_ROWS, 128), jnp.float32),
    mesh=_mesh,
    scratch_types=[
        pltpu.VMEM((2, EMBED_DIM, TPG * 128), jnp.float32),
        pltpu.VMEM((TPG * 64, 128), jnp.float32),
        pltpu.SemaphoreType.DMA,
        pltpu.SemaphoreType.DMA,
    ],
    compiler_params=pltpu.CompilerParams(needs_layout_passes=False),
)
def _sc_transpose(tableT_hbm, tail_hbm, t2_hbm, chunk_v, out_v, sem_in, sem_out):
    wid = lax.axis_index("s") * NC + lax.axis_index("c")
    lane = lax.iota(jnp.int32, 16)

    def start_in(g, buf):
        @pl.when(g < N_GROUPS)
        def _():
            col = pl.multiple_of(g * (TPG * 128), 128)
            pltpu.async_copy(
                tableT_hbm.at[:, pl.ds(col, TPG * 128)], chunk_v.at[buf], sem_in
            )

    start_in(wid, 0)

    def group(k, carry):
        g = wid + k * NW

        @pl.when(g < N_GROUPS)
        def _():
            for buf in range(2):

                @pl.when(k % 2 == buf)
                def _():
                    pltpu.make_async_copy(
                        tableT_hbm.at[:, pl.ds(0, TPG * 128)],
                        chunk_v.at[buf],
                        sem_in,
                    ).wait()
                    start_in(g + NW, 1 - buf)
                    @pl.when(k > 0)
                    def _():
                        pltpu.make_async_copy(
                            out_v, t2_hbm.at[pl.ds(0, TPG * 64)], sem_out
                        ).wait()

                    _transpose_group(chunk_v.at[buf], out_v, lane, TPG)
                    base = pl.multiple_of(g * (TPG * 64), 8)
                    pltpu.async_copy(
                        out_v, t2_hbm.at[pl.ds(base, TPG * 64)], sem_out
                    )

        return carry

    lax.fori_loop(0, G_PER_W, group, 0)

    pltpu.make_async_copy(out_v, t2_hbm.at[pl.ds(0, TPG * 64)], sem_out).wait()

    @pl.when(wid == 0)
    def _():
        pltpu.sync_copy(tail_hbm, chunk_v.at[0, :, pl.ds(0, 128)])
        _transpose_group(chunk_v.at[0], out_v, lane, 1)
        pltpu.sync_copy(
            out_v.at[pl.ds(0, 64)],
            t2_hbm.at[pl.ds((N_TILES - 1) * 64, 64)],
        )


@functools.partial(
    pl.kernel,
    out_type=jax.ShapeDtypeStruct((BATCH, 128), jnp.float32),
    mesh=_mesh,
    scratch_types=[
        pltpu.VMEM((N_CHUNKS, CHUNK), jnp.int32),
        pltpu.VMEM((B_PER_W, 128), jnp.float32),
        pltpu.SemaphoreType.DMA,
    ],
)
def _sc_gather(idx_hbm, table2_hbm, emb_hbm, idx_v, rows_v, sem):
    wid = lax.axis_index("s") * NC + lax.axis_index("c")
    base = wid * B_PER_W
    pltpu.sync_copy(idx_hbm.at[wid], idx_v)
    copies = [
        pltpu.async_copy(
            table2_hbm.at[idx_v.at[j]],
            rows_v.at[pl.ds(j * CHUNK, CHUNK)],
            sem,
        )
        for j in range(N_CHUNKS)
    ]
    for c in copies:
        c.wait()
    pltpu.sync_copy(rows_v, emb_hbm.at[pl.ds(base, B_PER_W)])


BM = 2048


def _mm_body(emb_ref, par_ref, ww_ref, b_ref, out_ref):
    lanes = lax.broadcasted_iota(jnp.int32, (BM, 128), 1)
    par = par_ref[...]
    sel = jnp.where(lanes < EMBED_DIM, 1.0 - par, par)
    out_ref[...] = (
        jnp.dot(emb_ref[...] * sel, ww_ref[...], preferred_element_type=jnp.float32)
        + b_ref[...]
    )


def _tc_linear(emb, par, WW, b2d):
    return pl.pallas_call(
        _mm_body,
        grid=(BATCH // BM,),
        in_specs=[
            pl.BlockSpec((BM, 128), lambda i: (i, 0)),
            pl.BlockSpec((BM, 1), lambda i: (i, 0)),
            pl.BlockSpec((128, OUT_DIM), lambda i: (0, 0)),
            pl.BlockSpec((1, OUT_DIM), lambda i: (0, 0)),
        ],
        out_specs=pl.BlockSpec((BM, OUT_DIM), lambda i: (i, 0)),
        out_shape=jax.ShapeDtypeStruct((BATCH, OUT_DIM), jnp.float32),
    )(emb, par, WW, b2d)


def kernel(x, table, W, b):
    xi = x.astype(jnp.int32)
    tableT = table.T
    tail = jnp.pad(
        lax.slice(tableT, (0, (N_TILES - 1) * 128), (EMBED_DIM, USER_DIM)),
        ((0, 0), (0, N_TILES * 128 - USER_DIM)),
    )
    table2 = _sc_transpose(tableT, tail)
    g = ((xi >> 7) << 6) | (xi & 63)
    idx2 = g.reshape(NW, N_CHUNKS, CHUNK)
    emb = _sc_gather(idx2, table2)
    par = ((xi >> 6) & 1).astype(jnp.float32).reshape(BATCH, 1)
    WW = jnp.concatenate([W, W], axis=0)
    return _tc_linear(emb, par, WW, b.reshape(1, OUT_DIM))


# speedup vs baseline: 6.2857x; 1.8912x over previous
import functools

import jax
import jax.numpy as jnp
from jax import lax
from jax.experimental import pallas as pl
from jax.experimental.pallas import tpu as pltpu
from jax.experimental.pallas import tpu_sc as plsc

USER_DIM = 1000000
EMBED_DIM = 64
OUT_DIM = 64
BATCH = 16384

NC, NS = 2, 16
NW = NC * NS
B_PER_W = BATCH // NW
CHUNK = 128
N_CHUNKS = B_PER_W // CHUNK

_mesh = plsc.VectorSubcoreMesh(
    core_axis_name="c", subcore_axis_name="s", num_cores=NC, num_subcores=NS
)


TBK = 8192


N_TBLK = (USER_DIM + TBK - 1) // TBK
T2_ROWS = N_TBLK * (TBK // 2)


def _tr_body(tT_ref, out_ref):
    t = jnp.swapaxes(tT_ref[...], 0, 1)
    out_ref[:, 0:EMBED_DIM] = t[0 : TBK // 2, :]
    out_ref[:, EMBED_DIM:128] = t[TBK // 2 : TBK, :]


def _tc_transpose(tableT):
    return pl.pallas_call(
        _tr_body,
        grid=(N_TBLK,),
        in_specs=[pl.BlockSpec((EMBED_DIM, TBK), lambda i: (0, i))],
        out_specs=pl.BlockSpec((TBK // 2, 128), lambda i: (i, 0)),
        out_shape=jax.ShapeDtypeStruct((T2_ROWS, 128), jnp.float32),
    )(tableT)


@functools.partial(
    pl.kernel,
    out_type=jax.ShapeDtypeStruct((BATCH, 128), jnp.float32),
    mesh=_mesh,
    scratch_types=[
        pltpu.VMEM((N_CHUNKS, CHUNK), jnp.int32),
        pltpu.VMEM((B_PER_W, 128), jnp.float32),
        pltpu.SemaphoreType.DMA,
    ],
)
def _sc_gather(idx_hbm, table2_hbm, emb_hbm, idx_v, rows_v, sem):
    wid = lax.axis_index("s") * NC + lax.axis_index("c")
    base = wid * B_PER_W
    pltpu.sync_copy(idx_hbm.at[wid], idx_v)
    copies = [
        pltpu.async_copy(
            table2_hbm.at[idx_v.at[j]],
            rows_v.at[pl.ds(j * CHUNK, CHUNK)],
            sem,
        )
        for j in range(N_CHUNKS)
    ]
    for c in copies:
        c.wait()
    pltpu.sync_copy(rows_v, emb_hbm.at[pl.ds(base, B_PER_W)])


BM = 2048


def _mm_body(emb_ref, par_ref, ww_ref, b_ref, out_ref):
    lanes = lax.broadcasted_iota(jnp.int32, (BM, 128), 1)
    par = par_ref[...]
    sel = jnp.where(lanes < EMBED_DIM, 1.0 - par, par)
    out_ref[...] = (
        jnp.dot(emb_ref[...] * sel, ww_ref[...], preferred_element_type=jnp.float32)
        + b_ref[...]
    )


def _tc_linear(emb, par, WW, b2d):
    return pl.pallas_call(
        _mm_body,
        grid=(BATCH // BM,),
        in_specs=[
            pl.BlockSpec((BM, 128), lambda i: (i, 0)),
            pl.BlockSpec((BM, 1), lambda i: (i, 0)),
            pl.BlockSpec((128, OUT_DIM), lambda i: (0, 0)),
            pl.BlockSpec((1, OUT_DIM), lambda i: (0, 0)),
        ],
        out_specs=pl.BlockSpec((BM, OUT_DIM), lambda i: (i, 0)),
        out_shape=jax.ShapeDtypeStruct((BATCH, OUT_DIM), jnp.float32),
    )(emb, par, WW, b2d)


def kernel(x, table, W, b):
    xi = x.astype(jnp.int32)
    half = TBK // 2
    g = (xi // TBK) * half + (xi % half)
    idx2 = g.reshape(NW, N_CHUNKS, CHUNK)
    table2 = _tc_transpose(table.T)
    emb = _sc_gather(idx2, table2)
    par = ((xi // half) & 1).astype(jnp.float32).reshape(BATCH, 1)
    WW = jnp.concatenate([W, W], axis=0)
    return _tc_linear(emb, par, WW, b.reshape(1, OUT_DIM))


# speedup vs baseline: 7.0678x; 1.1244x over previous
import functools

import jax
import jax.numpy as jnp
from jax import lax
from jax.experimental import pallas as pl
from jax.experimental.pallas import tpu as pltpu
from jax.experimental.pallas import tpu_sc as plsc

USER_DIM = 1000000
EMBED_DIM = 64
OUT_DIM = 64
BATCH = 16384

NC, NS = 2, 16
NW = NC * NS
B_PER_W = BATCH // NW
CHUNK = 128
N_CHUNKS = B_PER_W // CHUNK

_mesh = plsc.VectorSubcoreMesh(
    core_axis_name="c", subcore_axis_name="s", num_cores=NC, num_subcores=NS
)


TBK = 16384


N_TBLK = (USER_DIM + TBK - 1) // TBK
T2_ROWS = N_TBLK * (TBK // 2)


def _tr_body(tT_ref, out_ref):
    t = jnp.swapaxes(tT_ref[...], 0, 1)
    out_ref[:, 0:EMBED_DIM] = t[0 : TBK // 2, :]
    out_ref[:, EMBED_DIM:128] = t[TBK // 2 : TBK, :]


def _tc_transpose(tableT):
    return pl.pallas_call(
        _tr_body,
        grid=(N_TBLK,),
        in_specs=[pl.BlockSpec((EMBED_DIM, TBK), lambda i: (0, i))],
        out_specs=pl.BlockSpec((TBK // 2, 128), lambda i: (i, 0)),
        out_shape=jax.ShapeDtypeStruct((T2_ROWS, 128), jnp.float32),
    )(tableT)


@functools.partial(
    pl.kernel,
    out_type=jax.ShapeDtypeStruct((BATCH, 128), jnp.float32),
    mesh=_mesh,
    scratch_types=[
        pltpu.VMEM((N_CHUNKS, CHUNK), jnp.int32),
        pltpu.VMEM((B_PER_W, 128), jnp.float32),
        pltpu.SemaphoreType.DMA,
    ],
)
def _sc_gather(idx_hbm, table2_hbm, emb_hbm, idx_v, rows_v, sem):
    wid = lax.axis_index("s") * NC + lax.axis_index("c")
    base = wid * B_PER_W
    pltpu.sync_copy(idx_hbm.at[wid], idx_v)
    copies = [
        pltpu.async_copy(
            table2_hbm.at[idx_v.at[j]],
            rows_v.at[pl.ds(j * CHUNK, CHUNK)],
            sem,
        )
        for j in range(N_CHUNKS)
    ]
    for c in copies:
        c.wait()
    pltpu.sync_copy(rows_v, emb_hbm.at[pl.ds(base, B_PER_W)])


BM = 2048


def _mm_body(emb_ref, par_ref, ww_ref, b_ref, out_ref):
    lanes = lax.broadcasted_iota(jnp.int32, (BM, 128), 1)
    par = par_ref[...]
    sel = jnp.where(lanes < EMBED_DIM, 1.0 - par, par)
    out_ref[...] = (
        jnp.dot(emb_ref[...] * sel, ww_ref[...], preferred_element_type=jnp.float32)
        + b_ref[...]
    )


def _tc_linear(emb, par, WW, b2d):
    return pl.pallas_call(
        _mm_body,
        grid=(BATCH // BM,),
        in_specs=[
            pl.BlockSpec((BM, 128), lambda i: (i, 0)),
            pl.BlockSpec((BM, 1), lambda i: (i, 0)),
            pl.BlockSpec((128, OUT_DIM), lambda i: (0, 0)),
            pl.BlockSpec((1, OUT_DIM), lambda i: (0, 0)),
        ],
        out_specs=pl.BlockSpec((BM, OUT_DIM), lambda i: (i, 0)),
        out_shape=jax.ShapeDtypeStruct((BATCH, OUT_DIM), jnp.float32),
    )(emb, par, WW, b2d)


def kernel(x, table, W, b):
    xi = x.astype(jnp.int32)
    half = TBK // 2
    g = (xi // TBK) * half + (xi % half)
    idx2 = g.reshape(NW, N_CHUNKS, CHUNK)
    table2 = _tc_transpose(table.T)
    emb = _sc_gather(idx2, table2)
    par = ((xi // half) & 1).astype(jnp.float32).reshape(BATCH, 1)
    WW = jnp.concatenate([W, W], axis=0)
    return _tc_linear(emb, par, WW, b.reshape(1, OUT_DIM))
